# baseline (speedup 1.0000x reference)
import functools

import jax
import jax.numpy as jnp
from jax.experimental import pallas as pl
from jax.experimental.pallas import tpu as pltpu

_N_EMB = 1024
_EMB_DIM = 256
_BETA = 0.25
_N = 8192
_BLK = 1024


def _vq_body(fx_ref, e_ref, emb_ref, loss_ref):
    fx = fx_ref[...]
    emb_tab = e_ref[...]
    ones = jnp.ones((1, _EMB_DIM), jnp.float32)
    esq = jax.lax.dot_general(
        ones, emb_tab * emb_tab, (((1,), (1,)), ((), ())),
        preferred_element_type=jnp.float32)

    s = jax.lax.dot_general(
        fx, emb_tab, (((1,), (1,)), ((), ())),
        preferred_element_type=jnp.float32)
    xsq = jnp.sum(fx * fx, axis=1, keepdims=True)
    dist = (xsq + esq) - 2.0 * s

    mind = jnp.min(dist, axis=1, keepdims=True)
    iota = jax.lax.broadcasted_iota(jnp.int32, dist.shape, 1)
    idx = jnp.min(jnp.where(dist == mind, iota, _N_EMB),
                  axis=1, keepdims=True)
    onehot = (iota == idx).astype(jnp.float32)

    emb_ref[...] = jax.lax.dot_general(
        onehot, emb_tab, (((1,), (0,)), ((), ())),
        preferred_element_type=jnp.float32)

    @pl.when(pl.program_id(0) == 0)
    def _init():
        loss_ref[...] = jnp.zeros_like(loss_ref)

    loss_ref[...] += jnp.sum(mind, axis=(0, 1), keepdims=True)


@jax.jit
def _vq(fx, embeddings):
    return pl.pallas_call(
        _vq_body,
        grid=(_N // _BLK,),
        in_specs=[
            pl.BlockSpec((_BLK, _EMB_DIM), lambda i: (i, 0)),
            pl.BlockSpec((_N_EMB, _EMB_DIM), lambda i: (0, 0)),
        ],
        out_specs=[
            pl.BlockSpec((_BLK, _EMB_DIM), lambda i: (i, 0)),
            pl.BlockSpec((1, 1), lambda i: (0, 0)),
        ],
        out_shape=[
            jax.ShapeDtypeStruct((_N, _EMB_DIM), jnp.float32),
            jax.ShapeDtypeStruct((1, 1), jnp.float32),
        ],
    )(fx, embeddings)


def kernel(x, embeddings):
    b, c, h, w = x.shape
    fx = jnp.transpose(x, (0, 2, 3, 1)).reshape(b * h * w, c)
    fx = pltpu.with_memory_space_constraint(fx, pltpu.MemorySpace.HBM)
    emb_in = pltpu.with_memory_space_constraint(embeddings,
                                                pltpu.MemorySpace.HBM)
    emb_flat, loss_sum = _vq(fx, emb_in)
    emb = jnp.transpose(emb_flat.reshape(b, h, w, c), (0, 3, 1, 2))
    loss = loss_sum[0, 0] * ((1.0 + _BETA) / (b * c * h * w))
    return emb, loss
